# baseline (speedup 1.0000x reference)
import functools

import jax
import jax.numpy as jnp
from jax import lax
from jax.experimental import pallas as pl
from jax.experimental.pallas import tpu as pltpu
from jax.experimental.pallas import tpu_sc as plsc

N = 10000
H = 128
E = 320000
NP = 10112
NC = 2
NS = 16
CH = 128
NCH = 160
NBI = 16
RPT = NP // NS
IDXR = NS * NCH
EPAD = IDXR * CH
CW = 16

f32 = jnp.float32
i32 = jnp.int32


def _dot_t(x, w):
    return lax.dot_general(x, w, (((1,), (1,)), ((), ())),
                           preferred_element_type=f32)


def _make_segsum(W=H, nbi=NBI):
    mesh = plsc.VectorSubcoreMesh(core_axis_name="c", subcore_axis_name="s")
    out_type = [jax.ShapeDtypeStruct((NP, W), f32),
                jax.ShapeDtypeStruct((NP, W), f32)]
    scratch = [
        pltpu.VMEM((nbi, CH), i32),
        pltpu.VMEM((nbi, CH), i32),
        pltpu.VMEM((nbi, CH), i32),
        pltpu.VMEM((nbi, CH), i32),
        pltpu.VMEM((CH, W), f32),
        pltpu.VMEM((CH, W), f32),
        pltpu.VMEM((NCH,), i32),
        pltpu.VMEM_SHARED((NP, W), f32),
        pltpu.SemaphoreType.DMA,
        pltpu.SemaphoreType.DMA,
        pltpu.SemaphoreType.DMA,
        pltpu.SemaphoreType.DMA,
        pltpu.SemaphoreType.DMA,
        pltpu.SemaphoreType.DMA,
    ]

    NBLK = NCH // nbi

    def body(yu, yi, su, du, si, di, *rest):
        (acc_i_o, acc_u_o,
         idx_s0, idx_s1, idx_d0, idx_d1, rows0, rows1, rowidx, acc_sh,
         sem0, sem1, ssem0, ssem1, isem0, isem1) = rest

        cid = lax.axis_index("c")
        sid = lax.axis_index("s")
        r0 = sid * RPT

        def fill(r, _):
            for c in range(W // 16):
                rows0[r, pl.ds(c * 16, 16)] = jnp.zeros((16,), f32)
                rows1[r, pl.ds(c * 16, 16)] = jnp.zeros((16,), f32)
            return _
        lax.fori_loop(0, CH, fill, None)

        def fillr(k, _):
            rowidx[pl.ds(k * 16, 16)] = (sid * NCH + k * 16
                                         + lax.iota(i32, 16))
            return _
        lax.fori_loop(0, NCH // 16, fillr, None)

        for k in range(4):
            pltpu.sync_copy(rows0, acc_sh.at[pl.ds(r0 + k * CH, CH)])
        pltpu.sync_copy(rows0.at[pl.ds(0, RPT - 4 * CH)],
                        acc_sh.at[pl.ds(r0 + 4 * CH, RPT - 4 * CH)])
        plsc.subcore_barrier()

        rows = (rows0, rows1)
        sems = (sem0, sem1)
        ssems = (ssem0, ssem1)
        isems = (isem0, isem1)
        idx_ss = (idx_s0, idx_s1)
        idx_ds = (idx_d0, idx_d1)

        def run(ytab, s_hbm, d_hbm, acc_out):
            def stage(blk, pp):
                pltpu.async_copy(
                    s_hbm.at[rowidx.at[pl.ds(blk * nbi, nbi)]],
                    idx_ss[pp], isems[pp])
                pltpu.async_copy(
                    d_hbm.at[rowidx.at[pl.ds(blk * nbi, nbi)]],
                    idx_ds[pp], isems[pp])

            def stage_wait(pp):
                for buf in (idx_ss[pp], idx_ds[pp]):
                    pltpu.make_async_copy(
                        s_hbm.at[rowidx.at[pl.ds(0, nbi)]], buf,
                        isems[pp]).wait()

            def gissue(pp, j, b):
                for h in range(2):
                    pltpu.async_copy(
                        ytab.at[idx_ss[pp].at[j, pl.ds(h * 64, 64)]],
                        rows[b].at[pl.ds(h * 64, 64)], sems[b])

            def gwait(pp, j, b):
                for h in range(2):
                    pltpu.make_async_copy(
                        ytab.at[idx_ss[pp].at[j, pl.ds(h * 64, 64)]],
                        rows[b].at[pl.ds(h * 64, 64)], sems[b]).wait()

            def scat(pp, j, b):
                pltpu.async_copy(rows[b], acc_sh.at[idx_ds[pp].at[j]],
                                 ssems[b], add=True)

            def scat_wait(b):
                pltpu.make_async_copy(rows[b], acc_sh.at[idx_ds[0].at[0]],
                                      ssems[b]).wait()

            stage(0, 0)
            stage_wait(0)
            gissue(0, 0, 0)

            def bigblock(bp, carry):
                for p in range(2):
                    blk = bp * 2 + p

                    @pl.when(blk + 1 < NBLK)
                    def _prefetch():
                        stage(blk + 1, 1 - p)

                    def inner(j2, c):
                        for b in range(2):
                            j = j2 * 2 + b
                            gwait(p, j, b)
                            scat(p, j, b)

                            @pl.when(blk * nbi + j > 0)
                            def _wait_prev():
                                scat_wait(1 - b)
                            gissue(p, j + 1, 1 - b)
                        return c
                    lax.fori_loop(0, (nbi - 2) // 2, inner, None)

                    gwait(p, nbi - 2, 0)
                    scat(p, nbi - 2, 0)
                    scat_wait(1)
                    gissue(p, nbi - 1, 1)

                    gwait(p, nbi - 1, 1)
                    scat(p, nbi - 1, 1)

                    @pl.when(blk + 1 < NBLK)
                    def _bridge():
                        stage_wait(1 - p)
                        scat_wait(0)
                        gissue(1 - p, 0, 0)

                    @pl.when(blk + 1 >= NBLK)
                    def _tail_drain():
                        scat_wait(0)
                return carry
            lax.fori_loop(0, NBLK // 2, bigblock, None)
            scat_wait(1)

            plsc.subcore_barrier()
            for k in range(4):
                pltpu.sync_copy(acc_sh.at[pl.ds(r0 + k * CH, CH)],
                                acc_out.at[pl.ds(r0 + k * CH, CH)])
            pltpu.sync_copy(acc_sh.at[pl.ds(r0 + 4 * CH, RPT - 4 * CH)],
                            acc_out.at[pl.ds(r0 + 4 * CH, RPT - 4 * CH)])

        @pl.when(cid == 0)
        def _rel_u2i():
            run(yu, su, du, acc_i_o)

        @pl.when(cid == 1)
        def _rel_i2u():
            run(yi, si, di, acc_u_o)

    return pl.kernel(body, out_type=out_type, mesh=mesh,
                     scratch_types=scratch)


_segsum = _make_segsum()


def _make_count():
    mesh = plsc.VectorSubcoreMesh(core_axis_name="c", subcore_axis_name="s")
    out_type = [jax.ShapeDtypeStruct((NP, H), f32),
                jax.ShapeDtypeStruct((NP, H), f32)]
    scratch = [
        pltpu.VMEM((NBI, CH), i32),
        pltpu.VMEM((CH, H), f32),
        pltpu.VMEM((NCH,), i32),
        pltpu.VMEM_SHARED((NP, H), f32),
        pltpu.SemaphoreType.DMA,
        pltpu.SemaphoreType.DMA,
    ]

    def body(du, di, cnt_i_o, cnt_u_o, idx_d, ones, rowidx, cnt_sh,
             sem0, sem1):
        cid = lax.axis_index("c")
        sid = lax.axis_index("s")
        r0 = sid * RPT

        def fill(r, _):
            for c in range(H // 16):
                ones[r, pl.ds(c * 16, 16)] = jnp.zeros((16,), f32)
            return _
        lax.fori_loop(0, CH, fill, None)

        def fillr(k, _):
            rowidx[pl.ds(k * 16, 16)] = (sid * NCH + k * 16
                                         + lax.iota(i32, 16))
            return _
        lax.fori_loop(0, NCH // 16, fillr, None)

        for k in range(4):
            pltpu.sync_copy(ones, cnt_sh.at[pl.ds(r0 + k * CH, CH)])
        pltpu.sync_copy(ones.at[pl.ds(0, RPT - 4 * CH)],
                        cnt_sh.at[pl.ds(r0 + 4 * CH, RPT - 4 * CH)])

        def refill(r, _):
            for c in range(H // 16):
                ones[r, pl.ds(c * 16, 16)] = jnp.ones((16,), f32)
            return _
        lax.fori_loop(0, CH, refill, None)
        plsc.subcore_barrier()

        def run(d_hbm, cnt_out):
            def block(blk, carry):
                pltpu.async_copy(
                    d_hbm.at[rowidx.at[pl.ds(blk * NBI, NBI)]], idx_d,
                    sem0).wait()

                def inner(j, c):
                    pltpu.async_copy(ones, cnt_sh.at[idx_d.at[j]], sem1,
                                     add=True)
                    return c
                lax.fori_loop(0, NBI, inner, None)

                def drain(j, c):
                    pltpu.make_async_copy(ones, cnt_sh.at[idx_d.at[0]],
                                          sem1).wait()
                    return c
                lax.fori_loop(0, NBI, drain, None)
                return carry
            lax.fori_loop(0, NCH // NBI, block, None)

            plsc.subcore_barrier()
            for k in range(4):
                pltpu.sync_copy(cnt_sh.at[pl.ds(r0 + k * CH, CH)],
                                cnt_out.at[pl.ds(r0 + k * CH, CH)])
            pltpu.sync_copy(cnt_sh.at[pl.ds(r0 + 4 * CH, RPT - 4 * CH)],
                            cnt_out.at[pl.ds(r0 + 4 * CH, RPT - 4 * CH)])

        @pl.when(cid == 0)
        def _rel_u2i():
            run(du, cnt_i_o)

        @pl.when(cid == 1)
        def _rel_i2u():
            run(di, cnt_u_o)

    return pl.kernel(body, out_type=out_type, mesh=mesh,
                     scratch_types=scratch)


_count = _make_count()


_BLK = 1000
_GRID = N // _BLK

_row_spec = pl.BlockSpec((_BLK, H), lambda i: (i, 0))
_w_spec = pl.BlockSpec((H, H), lambda i: (0, 0))
_b_spec = pl.BlockSpec((1, H), lambda i: (0, 0))
_cnt_spec = pl.BlockSpec((_BLK, 1), lambda i: (i, 0))
_row_out = jax.ShapeDtypeStruct((N, H), f32)


def _lin_pair(xu, wu, xi, wi):
    def body(xu_r, xi_r, wu_r, wi_r, yu_r, yi_r):
        yu_r[...] = _dot_t(xu_r[...], wu_r[...])
        yi_r[...] = _dot_t(xi_r[...], wi_r[...])
    return pl.pallas_call(
        body, grid=(_GRID,),
        in_specs=[_row_spec, _row_spec, _w_spec, _w_spec],
        out_specs=[_row_spec, _row_spec],
        out_shape=[_row_out, _row_out],
    )(xu, xi, wu, wi)


def _finalize1(acc_i, cnt_i, x_i, wr_i, b_i, wl2_i,
               acc_u, cnt_u, x_u, wr_u, b_u, wl2_u):
    def body(ai, ci, xi, wri, bi, wli, au, cu, xu, wru, bu, wlu,
             zi_r, y2i_r, zu_r, y2u_r):
        def half(acc, cnt, x, wr, b, wl2, z_r, y2_r):
            mean = acc[...] / jnp.maximum(cnt[...], 1.0)
            z = jnp.maximum(mean + _dot_t(x[...], wr[...]) + b[...], 0.0)
            z_r[...] = z
            y2_r[...] = _dot_t(z, wl2[...])
        half(ai, ci, xi, wri, bi, wli, zi_r, y2i_r)
        half(au, cu, xu, wru, bu, wlu, zu_r, y2u_r)
    return pl.pallas_call(
        body, grid=(_GRID,),
        in_specs=[_row_spec, _cnt_spec, _row_spec, _w_spec, _b_spec, _w_spec,
                  _row_spec, _cnt_spec, _row_spec, _w_spec, _b_spec, _w_spec],
        out_specs=[_row_spec] * 4,
        out_shape=[_row_out] * 4,
    )(acc_i, cnt_i, x_i, wr_i, b_i, wl2_i,
      acc_u, cnt_u, x_u, wr_u, b_u, wl2_u)


def _finalize2(acc_i, cnt_i, z_i, wr_i, b_i,
               acc_u, cnt_u, z_u, wr_u, b_u):
    def body(ai, ci, zi, wri, bi, au, cu, zu, wru, bu, hi_r, hu_r):
        def half(acc, cnt, z, wr, b, h_r):
            mean = acc[...] / jnp.maximum(cnt[...], 1.0)
            h_r[...] = mean + _dot_t(z[...], wr[...]) + b[...]
        half(ai, ci, zi, wri, bi, hi_r)
        half(au, cu, zu, wru, bu, hu_r)
    return pl.pallas_call(
        body, grid=(_GRID,),
        in_specs=[_row_spec, _cnt_spec, _row_spec, _w_spec, _b_spec,
                  _row_spec, _cnt_spec, _row_spec, _w_spec, _b_spec],
        out_specs=[_row_spec] * 2,
        out_shape=[_row_out] * 2,
    )(acc_i, cnt_i, z_i, wr_i, b_i, acc_u, cnt_u, z_u, wr_u, b_u)


def _pad_idx(idx, fill):
    pad = jnp.full((EPAD - E,), fill, i32)
    return jnp.concatenate([idx.astype(i32), pad]).reshape(IDXR, CH)


def kernel(edge_index_u2i, edge_index_i2u, emb_user, emb_item,
           W1ui_l, W1ui_r, W1iu_l, W1iu_r, W2ui_l, W2ui_r, W2iu_l, W2iu_r,
           b1ui, b1iu, b2ui, b2iu):
    su = _pad_idx(edge_index_u2i[0], 0)
    du = _pad_idx(edge_index_u2i[1], N)
    si = _pad_idx(edge_index_i2u[0], 0)
    di = _pad_idx(edge_index_i2u[1], N)
    b1ui2 = b1ui.reshape(1, H)
    b1iu2 = b1iu.reshape(1, H)
    b2ui2 = b2ui.reshape(1, H)
    b2iu2 = b2iu.reshape(1, H)

    cnt_i, cnt_u = _count(du, di)
    cnt_i1 = cnt_i[:N, :1]
    cnt_u1 = cnt_u[:N, :1]

    y1u, y1i = _lin_pair(emb_user, W1ui_l, emb_item, W1iu_l)
    acc_i, acc_u = _segsum(y1u, y1i, su, du, si, di)
    z_i, y2i, z_u, y2u = _finalize1(
        acc_i[:N], cnt_i1, emb_item, W1ui_r, b1ui2, W2iu_l,
        acc_u[:N], cnt_u1, emb_user, W1iu_r, b1iu2, W2ui_l)

    acc2_i, acc2_u = _segsum(y2u, y2i, su, du, si, di)
    h_i, h_u = _finalize2(
        acc2_i[:N], cnt_i1, z_i, W2ui_r, b2ui2,
        acc2_u[:N], cnt_u1, z_u, W2iu_r, b2iu2)
    return (h_u, h_i)


# speedup vs baseline: 1.0133x; 1.0133x over previous
import jax
import jax.numpy as jnp
from jax import lax
from jax.experimental import pallas as pl
from jax.experimental.pallas import tpu as pltpu
from jax.experimental.pallas import tpu_sc as plsc

N = 10000
H = 128
E = 320000
NP = 10112
NC = 2
NS = 16
CH = 128
NCH = 160
NBI = 16
RPT = NP // NS
IDXR = NS * NCH
EPAD = IDXR * CH

f32 = jnp.float32
i32 = jnp.int32


def _dot_t(x, w):
    return lax.dot_general(x, w, (((1,), (1,)), ((), ())),
                           preferred_element_type=f32)


def _make_segsum(W=H, nbi=NBI):
    mesh = plsc.VectorSubcoreMesh(core_axis_name="c", subcore_axis_name="s")
    out_type = [jax.ShapeDtypeStruct((NP, W), f32),
                jax.ShapeDtypeStruct((NP, W), f32)]
    scratch = [
        pltpu.VMEM((nbi, CH), i32),
        pltpu.VMEM((nbi, CH), i32),
        pltpu.VMEM((nbi, CH), i32),
        pltpu.VMEM((nbi, CH), i32),
        pltpu.VMEM((CH, W), f32),
        pltpu.VMEM((CH, W), f32),
        pltpu.VMEM((NCH,), i32),
        pltpu.VMEM_SHARED((NP, W), f32),
        pltpu.SemaphoreType.DMA,
        pltpu.SemaphoreType.DMA,
        pltpu.SemaphoreType.DMA,
        pltpu.SemaphoreType.DMA,
        pltpu.SemaphoreType.DMA,
        pltpu.SemaphoreType.DMA,
    ]

    NBLK = NCH // nbi

    def body(yu, yi, su, du, si, di, *rest):
        (acc_i_o, acc_u_o,
         idx_s0, idx_s1, idx_d0, idx_d1, rows0, rows1, rowidx, acc_sh,
         sem0, sem1, ssem0, ssem1, isem0, isem1) = rest

        cid = lax.axis_index("c")
        sid = lax.axis_index("s")
        r0 = sid * RPT

        def fill(r, _):
            for c in range(W // 16):
                rows0[r, pl.ds(c * 16, 16)] = jnp.zeros((16,), f32)
                rows1[r, pl.ds(c * 16, 16)] = jnp.zeros((16,), f32)
            return _
        lax.fori_loop(0, CH, fill, None)

        def fillr(k, _):
            rowidx[pl.ds(k * 16, 16)] = (sid * NCH + k * 16
                                         + lax.iota(i32, 16))
            return _
        lax.fori_loop(0, NCH // 16, fillr, None)

        for k in range(4):
            pltpu.sync_copy(rows0, acc_sh.at[pl.ds(r0 + k * CH, CH)])
        pltpu.sync_copy(rows0.at[pl.ds(0, RPT - 4 * CH)],
                        acc_sh.at[pl.ds(r0 + 4 * CH, RPT - 4 * CH)])
        plsc.subcore_barrier()

        rows = (rows0, rows1)
        sems = (sem0, sem1)
        ssems = (ssem0, ssem1)
        isems = (isem0, isem1)
        idx_ss = (idx_s0, idx_s1)
        idx_ds = (idx_d0, idx_d1)

        def run(ytab, s_hbm, d_hbm, acc_out):
            def stage(blk, pp):
                pltpu.async_copy(
                    s_hbm.at[rowidx.at[pl.ds(blk * nbi, nbi)]],
                    idx_ss[pp], isems[pp])
                pltpu.async_copy(
                    d_hbm.at[rowidx.at[pl.ds(blk * nbi, nbi)]],
                    idx_ds[pp], isems[pp])

            def stage_wait(pp):
                for buf in (idx_ss[pp], idx_ds[pp]):
                    pltpu.make_async_copy(
                        s_hbm.at[rowidx.at[pl.ds(0, nbi)]], buf,
                        isems[pp]).wait()

            def gissue(pp, j, b):
                for h in range(2):
                    pltpu.async_copy(
                        ytab.at[idx_ss[pp].at[j, pl.ds(h * 64, 64)]],
                        rows[b].at[pl.ds(h * 64, 64)], sems[b])

            def gwait(pp, j, b):
                for h in range(2):
                    pltpu.make_async_copy(
                        ytab.at[idx_ss[pp].at[j, pl.ds(h * 64, 64)]],
                        rows[b].at[pl.ds(h * 64, 64)], sems[b]).wait()

            def scat(pp, j, b):
                pltpu.async_copy(rows[b], acc_sh.at[idx_ds[pp].at[j]],
                                 ssems[b], add=True)

            def scat_wait(b):
                pltpu.make_async_copy(rows[b], acc_sh.at[idx_ds[0].at[0]],
                                      ssems[b]).wait()

            stage(0, 0)
            stage_wait(0)
            gissue(0, 0, 0)

            def bigblock(bp, carry):
                for p in range(2):
                    blk = bp * 2 + p

                    @pl.when(blk + 1 < NBLK)
                    def _prefetch():
                        stage(blk + 1, 1 - p)

                    def inner(j2, c):
                        for b in range(2):
                            j = j2 * 2 + b
                            gwait(p, j, b)
                            scat(p, j, b)

                            @pl.when(blk * nbi + j > 0)
                            def _wait_prev():
                                scat_wait(1 - b)
                            gissue(p, j + 1, 1 - b)
                        return c
                    lax.fori_loop(0, (nbi - 2) // 2, inner, None)

                    gwait(p, nbi - 2, 0)
                    scat(p, nbi - 2, 0)
                    scat_wait(1)
                    gissue(p, nbi - 1, 1)

                    gwait(p, nbi - 1, 1)
                    scat(p, nbi - 1, 1)

                    @pl.when(blk + 1 < NBLK)
                    def _bridge():
                        stage_wait(1 - p)
                        scat_wait(0)
                        gissue(1 - p, 0, 0)

                    @pl.when(blk + 1 >= NBLK)
                    def _tail_drain():
                        scat_wait(0)
                return carry
            lax.fori_loop(0, NBLK // 2, bigblock, None)
            scat_wait(1)

            plsc.subcore_barrier()
            for k in range(4):
                pltpu.sync_copy(acc_sh.at[pl.ds(r0 + k * CH, CH)],
                                acc_out.at[pl.ds(r0 + k * CH, CH)])
            pltpu.sync_copy(acc_sh.at[pl.ds(r0 + 4 * CH, RPT - 4 * CH)],
                            acc_out.at[pl.ds(r0 + 4 * CH, RPT - 4 * CH)])

        @pl.when(cid == 0)
        def _rel_u2i():
            run(yu, su, du, acc_i_o)

        @pl.when(cid == 1)
        def _rel_i2u():
            run(yi, si, di, acc_u_o)

    return pl.kernel(body, out_type=out_type, mesh=mesh,
                     scratch_types=scratch)


_segsum = _make_segsum()


def _make_count():
    mesh = plsc.VectorSubcoreMesh(core_axis_name="c", subcore_axis_name="s")
    out_type = [jax.ShapeDtypeStruct((NP, H), f32),
                jax.ShapeDtypeStruct((NP, H), f32)]
    scratch = [
        pltpu.VMEM((NBI, CH), i32),
        pltpu.VMEM((CH, H), f32),
        pltpu.VMEM((NCH,), i32),
        pltpu.VMEM_SHARED((NP, H), f32),
        pltpu.SemaphoreType.DMA,
        pltpu.SemaphoreType.DMA,
    ]

    def body(du, di, cnt_i_o, cnt_u_o, idx_d, ones, rowidx, cnt_sh,
             sem0, sem1):
        cid = lax.axis_index("c")
        sid = lax.axis_index("s")
        r0 = sid * RPT

        def fill(r, _):
            for c in range(H // 16):
                ones[r, pl.ds(c * 16, 16)] = jnp.zeros((16,), f32)
            return _
        lax.fori_loop(0, CH, fill, None)

        def fillr(k, _):
            rowidx[pl.ds(k * 16, 16)] = (sid * NCH + k * 16
                                         + lax.iota(i32, 16))
            return _
        lax.fori_loop(0, NCH // 16, fillr, None)

        for k in range(4):
            pltpu.sync_copy(ones, cnt_sh.at[pl.ds(r0 + k * CH, CH)])
        pltpu.sync_copy(ones.at[pl.ds(0, RPT - 4 * CH)],
                        cnt_sh.at[pl.ds(r0 + 4 * CH, RPT - 4 * CH)])

        def refill(r, _):
            for c in range(H // 16):
                ones[r, pl.ds(c * 16, 16)] = jnp.ones((16,), f32)
            return _
        lax.fori_loop(0, CH, refill, None)
        plsc.subcore_barrier()

        def run(d_hbm, cnt_out):
            def block(blk, carry):
                pltpu.async_copy(
                    d_hbm.at[rowidx.at[pl.ds(blk * NBI, NBI)]], idx_d,
                    sem0).wait()

                def inner(j, c):
                    pltpu.async_copy(ones, cnt_sh.at[idx_d.at[j]], sem1,
                                     add=True)
                    return c
                lax.fori_loop(0, NBI, inner, None)

                def drain(j, c):
                    pltpu.make_async_copy(ones, cnt_sh.at[idx_d.at[0]],
                                          sem1).wait()
                    return c
                lax.fori_loop(0, NBI, drain, None)
                return carry
            lax.fori_loop(0, NCH // NBI, block, None)

            plsc.subcore_barrier()
            for k in range(4):
                pltpu.sync_copy(cnt_sh.at[pl.ds(r0 + k * CH, CH)],
                                cnt_out.at[pl.ds(r0 + k * CH, CH)])
            pltpu.sync_copy(cnt_sh.at[pl.ds(r0 + 4 * CH, RPT - 4 * CH)],
                            cnt_out.at[pl.ds(r0 + 4 * CH, RPT - 4 * CH)])

        @pl.when(cid == 0)
        def _rel_u2i():
            run(du, cnt_i_o)

        @pl.when(cid == 1)
        def _rel_i2u():
            run(di, cnt_u_o)

    return pl.kernel(body, out_type=out_type, mesh=mesh,
                     scratch_types=scratch)


_count = _make_count()


_BLK = 1000
_GRID = N // _BLK

_row_spec = pl.BlockSpec((_BLK, H), lambda i: (i, 0))
_w_spec = pl.BlockSpec((H, H), lambda i: (0, 0))
_b_spec = pl.BlockSpec((1, H), lambda i: (0, 0))
_cnt_spec = pl.BlockSpec((_BLK, 1), lambda i: (i, 0))
_row_out = jax.ShapeDtypeStruct((N, H), f32)


def _lin_pair(xu, wu, xi, wi):
    def body(xu_r, xi_r, wu_r, wi_r, yu_r, yi_r):
        yu_r[...] = _dot_t(xu_r[...], wu_r[...])
        yi_r[...] = _dot_t(xi_r[...], wi_r[...])
    return pl.pallas_call(
        body, grid=(_GRID,),
        in_specs=[_row_spec, _row_spec, _w_spec, _w_spec],
        out_specs=[_row_spec, _row_spec],
        out_shape=[_row_out, _row_out],
    )(xu, xi, wu, wi)


def _finalize1(acc_i, cnt_i, x_i, wr_i, b_i, wl2_i,
               acc_u, cnt_u, x_u, wr_u, b_u, wl2_u):
    def body(ai, ci, xi, wri, bi, wli, au, cu, xu, wru, bu, wlu,
             zi_r, y2i_r, zu_r, y2u_r):
        def half(acc, cnt, x, wr, b, wl2, z_r, y2_r):
            mean = acc[...] / jnp.maximum(cnt[...], 1.0)
            z = jnp.maximum(mean + _dot_t(x[...], wr[...]) + b[...], 0.0)
            z_r[...] = z
            y2_r[...] = _dot_t(z, wl2[...])
        half(ai, ci, xi, wri, bi, wli, zi_r, y2i_r)
        half(au, cu, xu, wru, bu, wlu, zu_r, y2u_r)
    return pl.pallas_call(
        body, grid=(_GRID,),
        in_specs=[_row_spec, _cnt_spec, _row_spec, _w_spec, _b_spec, _w_spec,
                  _row_spec, _cnt_spec, _row_spec, _w_spec, _b_spec, _w_spec],
        out_specs=[_row_spec] * 4,
        out_shape=[_row_out] * 4,
    )(acc_i, cnt_i, x_i, wr_i, b_i, wl2_i,
      acc_u, cnt_u, x_u, wr_u, b_u, wl2_u)


def _finalize2(acc_i, cnt_i, z_i, wr_i, b_i,
               acc_u, cnt_u, z_u, wr_u, b_u):
    def body(ai, ci, zi, wri, bi, au, cu, zu, wru, bu, hi_r, hu_r):
        def half(acc, cnt, z, wr, b, h_r):
            mean = acc[...] / jnp.maximum(cnt[...], 1.0)
            h_r[...] = mean + _dot_t(z[...], wr[...]) + b[...]
        half(ai, ci, zi, wri, bi, hi_r)
        half(au, cu, zu, wru, bu, hu_r)
    return pl.pallas_call(
        body, grid=(_GRID,),
        in_specs=[_row_spec, _cnt_spec, _row_spec, _w_spec, _b_spec,
                  _row_spec, _cnt_spec, _row_spec, _w_spec, _b_spec],
        out_specs=[_row_spec] * 2,
        out_shape=[_row_out] * 2,
    )(acc_i, cnt_i, z_i, wr_i, b_i, acc_u, cnt_u, z_u, wr_u, b_u)


def _pad_idx(idx, fill):
    pad = jnp.full((EPAD - E,), fill, i32)
    return jnp.concatenate([idx.astype(i32), pad]).reshape(IDXR, CH)


def kernel(edge_index_u2i, edge_index_i2u, emb_user, emb_item,
           W1ui_l, W1ui_r, W1iu_l, W1iu_r, W2ui_l, W2ui_r, W2iu_l, W2iu_r,
           b1ui, b1iu, b2ui, b2iu):
    su = _pad_idx(edge_index_u2i[0], 0)
    du = _pad_idx(edge_index_u2i[1], N)
    si = _pad_idx(edge_index_i2u[0], 0)
    di = _pad_idx(edge_index_i2u[1], N)
    b1ui2 = b1ui.reshape(1, H)
    b1iu2 = b1iu.reshape(1, H)
    b2ui2 = b2ui.reshape(1, H)
    b2iu2 = b2iu.reshape(1, H)

    cnt_i, cnt_u = _count(du, di)
    cnt_i1 = cnt_i[:N, :1]
    cnt_u1 = cnt_u[:N, :1]

    y1u, y1i = _lin_pair(emb_user, W1ui_l, emb_item, W1iu_l)
    acc_i, acc_u = _segsum(y1u, y1i, su, du, si, di)
    z_i, y2i, z_u, y2u = _finalize1(
        acc_i[:N], cnt_i1, emb_item, W1ui_r, b1ui2, W2iu_l,
        acc_u[:N], cnt_u1, emb_user, W1iu_r, b1iu2, W2ui_l)

    acc2_i, acc2_u = _segsum(y2u, y2i, su, du, si, di)
    h_i, h_u = _finalize2(
        acc2_i[:N], cnt_i1, z_i, W2ui_r, b2ui2,
        acc2_u[:N], cnt_u1, z_u, W2iu_r, b2iu2)
    return (h_u, h_i)
